# Initial kernel scaffold; baseline (speedup 1.0000x reference)
import jax
import jax.numpy as jnp
from jax.experimental import pallas as pl


def kernel(x, edge_index, W, a_l, a_r, bias):
    raise NotImplementedError("write your pallas kernel here")


# speedup vs baseline: 24.1068x; 24.1068x over previous
import functools

import jax
import jax.numpy as jnp
from jax import lax
from jax.experimental import pallas as pl
from jax.experimental.pallas import tpu as pltpu
from jax.experimental.pallas import tpu_sc as plsc

N = 10000
E = 320000
D = 128
NC = 2
NS = 16
NW = NC * NS
EP = E // NW
BLK = 128
NB = EP // BLK
TAIL = EP - NB * BLK
CH = 80
NCH = N // CH
NEG_SLOPE = 0.2


def _tc_prep_body(x_ref, wt_ref, al_ref, ar_ref, h_ref, el_ref, er_ref,
                  mr_ref):
    h = jnp.dot(x_ref[...], wt_ref[...], preferred_element_type=jnp.float32)
    h_ref[...] = h
    el_ref[...] = jnp.dot(h, al_ref[...], preferred_element_type=jnp.float32)
    er = jnp.dot(h, ar_ref[...], preferred_element_type=jnp.float32)
    er_ref[...] = er

    @pl.when(pl.program_id(0) == 0)
    def _():
        mr_ref[0, 0] = -jnp.inf

    mr_ref[0, 0] = jnp.maximum(mr_ref[0, 0], jnp.max(er))


_tc_prep = pl.pallas_call(
    _tc_prep_body,
    grid=(10,),
    in_specs=[
        pl.BlockSpec((N // 10, D), lambda i: (i, 0)),
        pl.BlockSpec((D, D), lambda i: (0, 0)),
        pl.BlockSpec((D, 1), lambda i: (0, 0)),
        pl.BlockSpec((D, 1), lambda i: (0, 0)),
    ],
    out_specs=[
        pl.BlockSpec((N // 10, D), lambda i: (i, 0)),
        pl.BlockSpec((N // 10, 1), lambda i: (i, 0)),
        pl.BlockSpec((N // 10, 1), lambda i: (i, 0)),
        pl.BlockSpec((1, 1), lambda i: (0, 0), memory_space=pltpu.SMEM),
    ],
    out_shape=[
        jax.ShapeDtypeStruct((N, D), jnp.float32),
        jax.ShapeDtypeStruct((N, 1), jnp.float32),
        jax.ShapeDtypeStruct((N, 1), jnp.float32),
        jax.ShapeDtypeStruct((1, 1), jnp.float32),
    ],
)


def _lrelu(z):
    return jnp.maximum(z, NEG_SLOPE * z)


def _sc_body(h_hbm, el_hbm, er_hbm, src_hbm, dst_hbm, mr_hbm,
             acc_out, s_out,
             el_v, er_v, s_v, src_v, dst_v, src_t, dst_t, p_v, rows_v,
             mr_v, acc_sh, sem):
    c = lax.axis_index("c")
    s = lax.axis_index("s")
    wid = s * NC + c

    pltpu.sync_copy(el_hbm, el_v)
    pltpu.sync_copy(er_hbm, er_v)
    pltpu.sync_copy(mr_hbm, mr_v)

    zero16 = jnp.zeros((16,), jnp.float32)

    def _zero_s(i, carry):
        s_v[pl.ds(i * 16, 16)] = zero16
        return carry

    lax.fori_loop(0, N // 16, _zero_s, 0)

    def _zero_rows(i, carry):
        for k in range(D // 16):
            rows_v[i, pl.ds(k * 16, 16)] = zero16
        return carry

    lax.fori_loop(0, BLK, _zero_rows, 0)

    nct = jnp.where(s < NCH % NS, NCH // NS + 1, NCH // NS)

    def _zero_chunk(i, carry):
        r0 = (s + i * NS) * CH
        pltpu.sync_copy(rows_v.at[pl.ds(0, CH)], acc_sh.at[pl.ds(r0, CH)])
        return carry

    lax.fori_loop(0, nct, _zero_chunk, 0)
    plsc.subcore_barrier()

    mr16 = mr_v[...]

    ebase = wid * EP

    def _edge_scalars(d16, s16, g):
        eld = plsc.load_gather(el_v, [d16])
        ers = plsc.load_gather(er_v, [s16])
        t = _lrelu(eld + ers)
        cm = _lrelu(eld + mr16)
        p = jnp.exp(t - cm)
        p_v[pl.ds(g * 16, 16)] = p
        plsc.addupdate_scatter(s_v, [d16], p)

    def _scale_row(j, carry):
        bc = plsc.load_gather(p_v, [jnp.full((16,), j, jnp.int32)])
        for k in range(D // 16):
            rows_v[j, pl.ds(k * 16, 16)] = rows_v[j, pl.ds(k * 16, 16)] * bc
        return carry

    def _block(b, carry):
        off = ebase + b * BLK
        pltpu.sync_copy(src_hbm.at[pl.ds(off, BLK)], src_v)
        pltpu.sync_copy(dst_hbm.at[pl.ds(off, BLK)], dst_v)
        pltpu.async_copy(h_hbm.at[src_v], rows_v, sem).wait()
        for g in range(BLK // 16):
            _edge_scalars(dst_v[pl.ds(g * 16, 16)], src_v[pl.ds(g * 16, 16)], g)
        lax.fori_loop(0, BLK, _scale_row, 0)
        pltpu.async_copy(rows_v, acc_sh.at[dst_v], sem, add=True).wait()
        return carry

    lax.fori_loop(0, NB, _block, 0)

    toff = ebase + NB * BLK
    pltpu.sync_copy(src_hbm.at[pl.ds(toff, TAIL)], src_t)
    pltpu.sync_copy(dst_hbm.at[pl.ds(toff, TAIL)], dst_t)
    pltpu.async_copy(h_hbm.at[src_t], rows_v.at[pl.ds(0, TAIL)], sem).wait()
    _edge_scalars(dst_t[...], src_t[...], 0)
    lax.fori_loop(0, TAIL, _scale_row, 0)
    pltpu.async_copy(rows_v.at[pl.ds(0, TAIL)], acc_sh.at[dst_t], sem,
                     add=True).wait()

    plsc.subcore_barrier()

    def _wb_chunk(i, carry):
        r0 = (s + i * NS) * CH
        pltpu.sync_copy(acc_sh.at[pl.ds(r0, CH)], rows_v.at[pl.ds(0, CH)])
        pltpu.sync_copy(rows_v.at[pl.ds(0, CH)],
                        acc_out.at[pl.ds(c * N + r0, CH)])
        return carry

    lax.fori_loop(0, nct, _wb_chunk, 0)
    pltpu.sync_copy(s_v, s_out.at[pl.ds(wid * N, N)])


_sc_edges = pl.kernel(
    _sc_body,
    out_type=[
        jax.ShapeDtypeStruct((NC * N, D), jnp.float32),
        jax.ShapeDtypeStruct((NW * N,), jnp.float32),
    ],
    mesh=plsc.VectorSubcoreMesh(core_axis_name="c", subcore_axis_name="s",
                                num_cores=NC, num_subcores=NS),
    compiler_params=pltpu.CompilerParams(needs_layout_passes=False),
    scratch_types=[
        pltpu.VMEM((N,), jnp.float32),
        pltpu.VMEM((N,), jnp.float32),
        pltpu.VMEM((N,), jnp.float32),
        pltpu.VMEM((BLK,), jnp.int32),
        pltpu.VMEM((BLK,), jnp.int32),
        pltpu.VMEM((TAIL,), jnp.int32),
        pltpu.VMEM((TAIL,), jnp.int32),
        pltpu.VMEM((BLK,), jnp.float32),
        pltpu.VMEM((BLK, D), jnp.float32),
        pltpu.VMEM((16,), jnp.float32),
        pltpu.VMEM_SHARED((N, D), jnp.float32),
        pltpu.SemaphoreType.DMA,
    ],
)


def _tc_norm_body(a0_ref, a1_ref, sp_ref, b_ref, o_ref):
    ssum = jnp.sum(sp_ref[...], axis=1)
    o_ref[...] = ((a0_ref[...] + a1_ref[...]) / (ssum + 1e-10)[:, None]
                  + b_ref[...])


_tc_norm = pl.pallas_call(
    _tc_norm_body,
    grid=(10,),
    in_specs=[
        pl.BlockSpec((N // 10, D), lambda i: (i, 0)),
        pl.BlockSpec((N // 10, D), lambda i: (i, 0)),
        pl.BlockSpec((N // 10, NW), lambda i: (i, 0)),
        pl.BlockSpec((1, D), lambda i: (0, 0)),
    ],
    out_specs=pl.BlockSpec((N // 10, D), lambda i: (i, 0)),
    out_shape=jax.ShapeDtypeStruct((N, D), jnp.float32),
)


@jax.jit
def kernel(x, edge_index, W, a_l, a_r, bias):
    h, el, er, mr = _tc_prep(x, W.T, a_l.reshape(D, 1), a_r.reshape(D, 1))
    src = edge_index[0]
    dst = edge_index[1]
    mr16 = jnp.broadcast_to(mr.reshape(1), (16,))
    acc, sp = _sc_edges(h, el.reshape(-1), er.reshape(-1), src, dst, mr16)
    return _tc_norm(acc[:N], acc[N:], sp.reshape(NW, N).T, bias.reshape(1, D))


# speedup vs baseline: 34.0605x; 1.4129x over previous
import jax
import jax.numpy as jnp
from jax import lax
from jax.experimental import pallas as pl
from jax.experimental.pallas import tpu as pltpu
from jax.experimental.pallas import tpu_sc as plsc

N = 10000
E = 320000
D = 128
NC = 2
NS = 16
NW = NC * NS
BLK = 128
NBLK = E // BLK
NBLKP = 2504
BPT = 80
CHK = 8
CH = 80
NCH = N // CH
NEG_SLOPE = 0.2


def _tc_prep_body(x_ref, wt_ref, al_ref, ar_ref, h_ref, el_ref, er_ref,
                  mr_ref):
    h = jnp.dot(x_ref[...], wt_ref[...], preferred_element_type=jnp.float32)
    h_ref[...] = h
    el_ref[...] = jnp.dot(h, al_ref[...], preferred_element_type=jnp.float32)
    er = jnp.dot(h, ar_ref[...], preferred_element_type=jnp.float32)
    er_ref[...] = er

    @pl.when(pl.program_id(0) == 0)
    def _():
        mr_ref[0, 0] = -jnp.inf

    mr_ref[0, 0] = jnp.maximum(mr_ref[0, 0], jnp.max(er))


_tc_prep = pl.pallas_call(
    _tc_prep_body,
    grid=(10,),
    in_specs=[
        pl.BlockSpec((N // 10, D), lambda i: (i, 0)),
        pl.BlockSpec((D, D), lambda i: (0, 0)),
        pl.BlockSpec((D, 1), lambda i: (0, 0)),
        pl.BlockSpec((D, 1), lambda i: (0, 0)),
    ],
    out_specs=[
        pl.BlockSpec((N // 10, D), lambda i: (i, 0)),
        pl.BlockSpec((N // 10, 1), lambda i: (i, 0)),
        pl.BlockSpec((N // 10, 1), lambda i: (i, 0)),
        pl.BlockSpec((1, 1), lambda i: (0, 0), memory_space=pltpu.SMEM),
    ],
    out_shape=[
        jax.ShapeDtypeStruct((N, D), jnp.float32),
        jax.ShapeDtypeStruct((N, 1), jnp.float32),
        jax.ShapeDtypeStruct((N, 1), jnp.float32),
        jax.ShapeDtypeStruct((1, 1), jnp.float32),
    ],
)


def _lrelu(z):
    return jnp.maximum(z, NEG_SLOPE * z)


def _tile_nb(wid):
    return jnp.where(wid < NW - 1, BPT, NBLKP - (NW - 1) * BPT)


def _sc_attn_body(el_hbm, er_hbm, src_hbm, dst_hbm, mr_hbm,
                  p_out, s_out,
                  el_v, er_v, s_v, srcc, dstc, pc, mr_v):
    c = lax.axis_index("c")
    s = lax.axis_index("s")
    wid = s * NC + c

    pltpu.sync_copy(el_hbm, el_v)
    pltpu.sync_copy(er_hbm, er_v)
    pltpu.sync_copy(mr_hbm, mr_v)

    zero16 = jnp.zeros((16,), jnp.float32)

    def _zero_s(i, carry):
        s_v[pl.ds(i * 16, 16)] = zero16
        return carry

    lax.fori_loop(0, N // 16, _zero_s, 0)

    mr16 = mr_v[...]
    nchunk = _tile_nb(wid) // CHK

    def _chunk(ci, carry):
        gb = wid * BPT + ci * CHK
        pltpu.sync_copy(src_hbm.at[pl.ds(gb, CHK)], srcc)
        pltpu.sync_copy(dst_hbm.at[pl.ds(gb, CHK)], dstc)

        def _blockp(j, carry2):
            valid = (gb + j) < NBLK
            for g in range(BLK // 16):
                d16 = dstc[j, pl.ds(g * 16, 16)]
                s16 = srcc[j, pl.ds(g * 16, 16)]
                eld = plsc.load_gather(el_v, [d16])
                ers = plsc.load_gather(er_v, [s16])
                t = _lrelu(eld + ers)
                cm = _lrelu(eld + mr16)
                p = jnp.where(valid, jnp.exp(t - cm), zero16)
                pc[j, pl.ds(g * 16, 16)] = p
                plsc.addupdate_scatter(s_v, [d16], p)
            return carry2

        lax.fori_loop(0, CHK, _blockp, 0)
        pltpu.sync_copy(pc, p_out.at[pl.ds(gb, CHK)])
        return carry

    lax.fori_loop(0, nchunk, _chunk, 0)
    pltpu.sync_copy(s_v, s_out.at[pl.ds(wid * N, N)])


_sc_attn = pl.kernel(
    _sc_attn_body,
    out_type=[
        jax.ShapeDtypeStruct((NBLKP, BLK), jnp.float32),
        jax.ShapeDtypeStruct((NW * N,), jnp.float32),
    ],
    mesh=plsc.VectorSubcoreMesh(core_axis_name="c", subcore_axis_name="s",
                                num_cores=NC, num_subcores=NS),
    compiler_params=pltpu.CompilerParams(needs_layout_passes=False),
    scratch_types=[
        pltpu.VMEM((N,), jnp.float32),
        pltpu.VMEM((N,), jnp.float32),
        pltpu.VMEM((N,), jnp.float32),
        pltpu.VMEM((CHK, BLK), jnp.int32),
        pltpu.VMEM((CHK, BLK), jnp.int32),
        pltpu.VMEM((CHK, BLK), jnp.float32),
        pltpu.VMEM((16,), jnp.float32),
    ],
)


def _sc_agg_body(h_hbm, src_hbm, dst_hbm, p_hbm,
                 acc_out,
                 srcc, dstc, pc, buf_a, buf_b,
                 acc_sh, gs_a, gs_b, ss_a, ss_b):
    c = lax.axis_index("c")
    s = lax.axis_index("s")
    wid = s * NC + c

    zero16 = jnp.zeros((16,), jnp.float32)

    def _zero_rows(i, carry):
        for k in range(D // 16):
            buf_a[i, pl.ds(k * 16, 16)] = zero16
        return carry

    lax.fori_loop(0, BLK, _zero_rows, 0)

    nct = jnp.where(s < NCH % NS, NCH // NS + 1, NCH // NS)

    def _zero_chunk(i, carry):
        r0 = (s + i * NS) * CH
        pltpu.sync_copy(buf_a.at[pl.ds(0, CH)], acc_sh.at[pl.ds(r0, CH)])
        return carry

    lax.fori_loop(0, nct, _zero_chunk, 0)
    plsc.subcore_barrier()

    def _scale(buf, j):
        def _rows(r2, carry):
            for u in range(2):
                r = r2 * 2 + u
                bc = plsc.load_gather(
                    pc, [jnp.full((16,), j, jnp.int32),
                         jnp.full((16,), r, jnp.int32)])
                for k in range(D // 16):
                    buf[r, pl.ds(k * 16, 16)] = buf[r, pl.ds(k * 16, 16)] * bc
            return carry

        lax.fori_loop(0, BLK // 2, _rows, 0)

    def _issue_g(j, buf, sem):
        pltpu.async_copy(h_hbm.at[srcc.at[j]], buf, sem)

    def _wait_g(j, buf, sem):
        pltpu.make_async_copy(h_hbm.at[srcc.at[j]], buf, sem).wait()

    def _issue_s(j, buf, sem):
        pltpu.async_copy(buf, acc_sh.at[dstc.at[j]], sem, add=True)

    def _wait_s(j, buf, sem):
        pltpu.make_async_copy(buf, acc_sh.at[dstc.at[j]], sem).wait()

    nchunk = _tile_nb(wid) // CHK

    def _chunk(ci, carry):
        gb = wid * BPT + ci * CHK
        pltpu.sync_copy(src_hbm.at[pl.ds(gb, CHK)], srcc)
        pltpu.sync_copy(dst_hbm.at[pl.ds(gb, CHK)], dstc)
        pltpu.sync_copy(p_hbm.at[pl.ds(gb, CHK)], pc)
        _issue_g(0, buf_a, gs_a)

        def _pair(j, carry2):
            b0 = j * 2
            b1 = b0 + 1
            _wait_g(b0, buf_a, gs_a)

            @pl.when(j > 0)
            def _():
                _wait_s(b1 - 2, buf_b, ss_b)

            _issue_g(b1, buf_b, gs_b)
            _scale(buf_a, b0)
            _issue_s(b0, buf_a, ss_a)
            _wait_g(b1, buf_b, gs_b)

            @pl.when(j < CHK // 2 - 1)
            def _():
                _wait_s(b0, buf_a, ss_a)
                _issue_g(b0 + 2, buf_a, gs_a)

            _scale(buf_b, b1)
            _issue_s(b1, buf_b, ss_b)
            return carry2

        lax.fori_loop(0, CHK // 2, _pair, 0)
        _wait_s(CHK - 2, buf_a, ss_a)
        _wait_s(CHK - 1, buf_b, ss_b)
        return carry

    lax.fori_loop(0, nchunk, _chunk, 0)
    plsc.subcore_barrier()

    def _wb_chunk(i, carry):
        r0 = (s + i * NS) * CH
        pltpu.sync_copy(acc_sh.at[pl.ds(r0, CH)], buf_a.at[pl.ds(0, CH)])
        pltpu.sync_copy(buf_a.at[pl.ds(0, CH)],
                        acc_out.at[pl.ds(c * N + r0, CH)])
        return carry

    lax.fori_loop(0, nct, _wb_chunk, 0)


_sc_agg = pl.kernel(
    _sc_agg_body,
    out_type=jax.ShapeDtypeStruct((NC * N, D), jnp.float32),
    mesh=plsc.VectorSubcoreMesh(core_axis_name="c", subcore_axis_name="s",
                                num_cores=NC, num_subcores=NS),
    compiler_params=pltpu.CompilerParams(needs_layout_passes=False),
    scratch_types=[
        pltpu.VMEM((CHK, BLK), jnp.int32),
        pltpu.VMEM((CHK, BLK), jnp.int32),
        pltpu.VMEM((CHK, BLK), jnp.float32),
        pltpu.VMEM((BLK, D), jnp.float32),
        pltpu.VMEM((BLK, D), jnp.float32),
        pltpu.VMEM_SHARED((N, D), jnp.float32),
        pltpu.SemaphoreType.DMA,
        pltpu.SemaphoreType.DMA,
        pltpu.SemaphoreType.DMA,
        pltpu.SemaphoreType.DMA,
    ],
)


def _tc_norm_body(a0_ref, a1_ref, sp_ref, b_ref, o_ref):
    ssum = jnp.sum(sp_ref[...], axis=1)
    o_ref[...] = ((a0_ref[...] + a1_ref[...]) / (ssum + 1e-10)[:, None]
                  + b_ref[...])


_tc_norm = pl.pallas_call(
    _tc_norm_body,
    grid=(10,),
    in_specs=[
        pl.BlockSpec((N // 10, D), lambda i: (i, 0)),
        pl.BlockSpec((N // 10, D), lambda i: (i, 0)),
        pl.BlockSpec((N // 10, NW), lambda i: (i, 0)),
        pl.BlockSpec((1, D), lambda i: (0, 0)),
    ],
    out_specs=pl.BlockSpec((N // 10, D), lambda i: (i, 0)),
    out_shape=jax.ShapeDtypeStruct((N, D), jnp.float32),
)


@jax.jit
def kernel(x, edge_index, W, a_l, a_r, bias):
    h, el, er, mr = _tc_prep(x, W.T, a_l.reshape(D, 1), a_r.reshape(D, 1))
    pad = jnp.zeros(((NBLKP - NBLK) * BLK,), jnp.int32)
    src = jnp.concatenate([edge_index[0], pad]).reshape(NBLKP, BLK)
    dst = jnp.concatenate([edge_index[1], pad]).reshape(NBLKP, BLK)
    mr16 = jnp.broadcast_to(mr.reshape(1), (16,))
    p_all, sp = _sc_attn(el.reshape(-1), er.reshape(-1), src, dst, mr16)
    acc = _sc_agg(h, src, dst, p_all)
    return _tc_norm(acc[:N], acc[N:], sp.reshape(NW, N).T, bias.reshape(1, D))
